# baseline (speedup 1.0000x reference)
import functools

import jax
import jax.numpy as jnp
from jax import lax
from jax.experimental import pallas as pl
from jax.experimental.pallas import tpu as pltpu
from jax.experimental.pallas import tpu_sc as plsc

B = 16384
D = 64
NC = 2
NS = 16
L = 16
NW = NC * NS
BPW = B // NW
TW = 128
TB = 256
PADT = 1_000_064
NSLOT = 3
GCH = 128
NG = BPW // GCH


@functools.partial(
    pl.kernel,
    out_type=jax.ShapeDtypeStruct((B, TW), jnp.float32),
    mesh=plsc.VectorSubcoreMesh(core_axis_name="c", subcore_axis_name="s",
                                num_cores=NC, num_subcores=NS),
    scratch_types=[
        pltpu.VMEM((BPW + L,), jnp.int32),
        pltpu.VMEM((BPW + L,), jnp.int32),
        pltpu.VMEM((BPW + L,), jnp.int32),
        pltpu.VMEM((BPW + L,), jnp.int32),
        pltpu.VMEM((NG, GCH), jnp.int32),
        pltpu.VMEM((NG, GCH), jnp.int32),
        pltpu.VMEM((NSLOT, D, TB), jnp.float32),
        pltpu.VMEM((BPW, TW), jnp.float32),
        pltpu.SemaphoreType.DMA,
        pltpu.SemaphoreType.DMA,
    ],
    compiler_params=pltpu.CompilerParams(needs_layout_passes=False),
)
def _emb_mul(blk_hbm, lane_hbm, ns_hbm, hl_hbm, ord2_hbm, ordr_hbm, lat_hbm,
             tab_hbm, out_hbm,
             blk_v, lane_v, ns_v, hl_v, ord2_v, ordr_v, ring_v, lat_v,
             gsem, lsem):
    wid = lax.axis_index("s") * NC + lax.axis_index("c")
    base = wid * BPW

    pltpu.sync_copy(blk_hbm.at[pl.ds(base, BPW)], blk_v.at[pl.ds(0, BPW)])
    pltpu.sync_copy(lane_hbm.at[pl.ds(base, BPW)], lane_v.at[pl.ds(0, BPW)])
    pltpu.sync_copy(ns_hbm.at[pl.ds(base, BPW)], ns_v.at[pl.ds(0, BPW)])
    pltpu.sync_copy(hl_hbm.at[pl.ds(base, BPW)], hl_v.at[pl.ds(0, BPW)])
    pltpu.sync_copy(ord2_hbm.at[wid], ord2_v)
    pltpu.sync_copy(ordr_hbm.at[wid], ordr_v)
    lat_cps = [
        pltpu.async_copy(lat_hbm.at[ord2_v.at[g]],
                         lat_v.at[pl.ds(g * GCH, GCH)], lsem)
        for g in range(NG)
    ]

    lanes = lax.iota(jnp.int32, L)

    def rd(ref, lidx):
        c = jnp.clip(lidx, 0, BPW - 1)
        return ref[pl.ds(c, L)][0]

    def fire(qg, slot):
        ql = qg - base
        blk = rd(blk_v, ql)

        @pl.when(jnp.logical_and(ql >= 0, ql < BPW))
        def _():
            pltpu.async_copy(
                tab_hbm.at[:, pl.ds(pl.multiple_of(blk, TW), TB)],
                ring_v.at[slot], gsem)

    def chase(qg):
        nxt = rd(ns_v, qg - base)
        return jnp.where(qg - base < BPW, nxt, jnp.int32(B))

    q = jnp.int32(base)
    for f0 in range(NSLOT):
        fire(q, f0)
        q = chase(q)
    for cp in lat_cps:
        cp.wait()

    def outer_cond(c):
        return c[0] < BPW

    def outer(c):
        p, f, q = c
        slot = lax.rem(f, NSLOT)
        pltpu.make_async_copy(tab_hbm.at[:, pl.ds(0, TB)], ring_v.at[slot],
                              gsem).wait()
        ssp = jnp.full((L,), slot, jnp.int32)
        run_end = jnp.minimum(rd(ns_v, p) - base, BPW)

        def consume(p2, carry):
            lsp = jnp.full((L,), rd(lane_v, p2), jnp.int32)
            psp = jnp.full((L,), p2, jnp.int32)
            ho = jnp.full((L,), rd(hl_v, p2), jnp.int32)
            for j in range(D // L):
                col = j * L + lanes
                v = plsc.load_gather(ring_v, [ssp, col, lsp])
                lv = plsc.load_gather(lat_v, [psp, ho + col])
                lat_v[p2, pl.ds(j * L, L)] = v * lv
            return carry

        lax.fori_loop(p, run_end, consume, 0)
        fire(q, slot)
        return (run_end, f + 1, chase(q))

    lax.while_loop(outer_cond, outer, (jnp.int32(0), jnp.int32(0), q))

    scat = [
        pltpu.async_copy(lat_v.at[pl.ds(g * GCH, GCH)],
                         out_hbm.at[ordr_v.at[g]], lsem)
        for g in range(NG)
    ]
    for cp in scat:
        cp.wait()


def kernel(label, latent, emb_table):
    lab = label.astype(jnp.int32)
    iot = lax.iota(jnp.int32, B)
    sl, order = lax.sort_key_val(lab, iot)
    blk = jnp.minimum((sl // TB) * TB, PADT - TB)
    lane_s = sl - blk
    newb = jnp.concatenate([jnp.ones((1,), jnp.bool_), blk[1:] != blk[:-1]])
    start_idx = jnp.where(newb, iot, B)
    ns_incl = jnp.flip(lax.cummin(jnp.flip(start_idx)))
    ns = jnp.concatenate([ns_incl[1:], jnp.full((1,), B, jnp.int32)])
    latp = latent.reshape(B // 2, TW)
    hl = (order & 1) * D
    ord2 = (order >> 1).reshape(NW, NG, GCH)
    ordr = order.reshape(NW, NG, GCH)
    out3 = _emb_mul(blk, lane_s, ns, hl, ord2, ordr, latp, emb_table.T)
    return out3[:, :D]


# speedup vs baseline: 1.0986x; 1.0986x over previous
import functools

import jax
import jax.numpy as jnp
from jax import lax
from jax.experimental import pallas as pl
from jax.experimental.pallas import tpu as pltpu
from jax.experimental.pallas import tpu_sc as plsc

B = 16384
D = 64
NC = 2
NS = 16
L = 16
NW = NC * NS
BPW = B // NW
TW = 128
NSLOT = 6
GCH = 128
NG = BPW // GCH


@functools.partial(
    pl.kernel,
    out_type=jax.ShapeDtypeStruct((B, TW), jnp.float32),
    mesh=plsc.VectorSubcoreMesh(core_axis_name="c", subcore_axis_name="s",
                                num_cores=NC, num_subcores=NS),
    scratch_types=[
        pltpu.VMEM((BPW + L,), jnp.int32),
        pltpu.VMEM((BPW + L,), jnp.int32),
        pltpu.VMEM((BPW + L,), jnp.int32),
        pltpu.VMEM((BPW + L,), jnp.int32),
        pltpu.VMEM((NG, GCH), jnp.int32),
        pltpu.VMEM((NG, GCH), jnp.int32),
        pltpu.VMEM((NSLOT, D, TW), jnp.float32),
        pltpu.VMEM((BPW, TW), jnp.float32),
        pltpu.SemaphoreType.DMA,
        pltpu.SemaphoreType.DMA,
    ],
    compiler_params=pltpu.CompilerParams(needs_layout_passes=False),
)
def _emb_mul(blk_hbm, lane_hbm, ns_hbm, hl_hbm, ord2_hbm, ordr_hbm, lat_hbm,
             tab_hbm, out_hbm,
             blk_v, lane_v, ns_v, hl_v, ord2_v, ordr_v, ring_v, lat_v,
             gsem, lsem):
    wid = lax.axis_index("s") * NC + lax.axis_index("c")
    base = wid * BPW

    pltpu.sync_copy(blk_hbm.at[pl.ds(base, BPW)], blk_v.at[pl.ds(0, BPW)])
    pltpu.sync_copy(lane_hbm.at[pl.ds(base, BPW)], lane_v.at[pl.ds(0, BPW)])
    pltpu.sync_copy(ns_hbm.at[pl.ds(base, BPW)], ns_v.at[pl.ds(0, BPW)])
    pltpu.sync_copy(hl_hbm.at[pl.ds(base, BPW)], hl_v.at[pl.ds(0, BPW)])
    pltpu.sync_copy(ord2_hbm.at[wid], ord2_v)
    pltpu.sync_copy(ordr_hbm.at[wid], ordr_v)
    lat_cps = [
        pltpu.async_copy(lat_hbm.at[ord2_v.at[g]],
                         lat_v.at[pl.ds(g * GCH, GCH)], lsem)
        for g in range(NG)
    ]

    lanes = lax.iota(jnp.int32, L)

    def rd(ref, lidx):
        c = jnp.clip(lidx, 0, BPW - 1)
        return ref[pl.ds(c, L)][0]

    def fire(qg, slot):
        ql = qg - base
        blk = rd(blk_v, ql)

        @pl.when(jnp.logical_and(ql >= 0, ql < BPW))
        def _():
            pltpu.async_copy(
                tab_hbm.at[:, pl.ds(pl.multiple_of(blk * TW, TW), TW)],
                ring_v.at[slot], gsem)

    def chase(qg):
        nxt = rd(ns_v, qg - base)
        return jnp.where(qg - base < BPW, nxt, jnp.int32(B))

    q = jnp.int32(base)
    for f0 in range(NSLOT):
        fire(q, f0)
        q = chase(q)
    for cp in lat_cps:
        cp.wait()

    def outer_cond(c):
        return c[0] < BPW

    def outer(c):
        p, f, q = c
        slot = lax.rem(f, NSLOT)
        pltpu.make_async_copy(tab_hbm.at[:, pl.ds(0, TW)], ring_v.at[slot],
                              gsem).wait()
        ssp = jnp.full((L,), slot, jnp.int32)
        run_end = jnp.minimum(rd(ns_v, p) - base, BPW)

        def consume(p2, carry):
            lsp = jnp.full((L,), rd(lane_v, p2), jnp.int32)
            psp = jnp.full((L,), p2, jnp.int32)
            ho = jnp.full((L,), rd(hl_v, p2), jnp.int32)
            for j in range(D // L):
                col = j * L + lanes
                v = plsc.load_gather(ring_v, [ssp, col, lsp])
                lv = plsc.load_gather(lat_v, [psp, ho + col])
                lat_v[p2, pl.ds(j * L, L)] = v * lv
            return carry

        lax.fori_loop(p, run_end, consume, 0)
        fire(q, slot)
        return (run_end, f + 1, chase(q))

    lax.while_loop(outer_cond, outer, (jnp.int32(0), jnp.int32(0), q))

    scat = [
        pltpu.async_copy(lat_v.at[pl.ds(g * GCH, GCH)],
                         out_hbm.at[ordr_v.at[g]], lsem)
        for g in range(NG)
    ]
    for cp in scat:
        cp.wait()


def kernel(label, latent, emb_table):
    lab = label.astype(jnp.int32)
    iot = lax.iota(jnp.int32, B)
    sl, order = lax.sort_key_val(lab, iot)
    blk = sl >> 7
    lane_s = sl & (TW - 1)
    newb = jnp.concatenate([jnp.ones((1,), jnp.bool_), blk[1:] != blk[:-1]])
    start_idx = jnp.where(newb, iot, B)
    ns_incl = jnp.flip(lax.cummin(jnp.flip(start_idx)))
    ns = jnp.concatenate([ns_incl[1:], jnp.full((1,), B, jnp.int32)])
    latp = latent.reshape(B // 2, TW)
    hl = (order & 1) * D
    ord2 = (order >> 1).reshape(NW, NG, GCH)
    ordr = order.reshape(NW, NG, GCH)
    out3 = _emb_mul(blk, lane_s, ns, hl, ord2, ordr, latp, emb_table.T)
    return out3[:, :D]
